# baseline (speedup 1.0000x reference)
import jax
import jax.numpy as jnp
from jax import lax
from jax.experimental import pallas as pl
from jax.experimental.pallas import tpu as pltpu
from jax.experimental.pallas import tpu_sc as plsc

N = 10000
E = 320000
D = 128

NC = 2
NS = 16
NW = NC * NS

CHUNK = 80
CPS = 25
NSUP = 5
EDGES_PER_TILE = CHUNK * CPS * NSUP
N_PAD = N
ROWS_MAIN = 624
REM_BASE = NS * ROWS_MAIN
REM = N - REM_BASE


def _mm_body(x_ref, w_ref, a_ref, wh_ref, s_ref):
    xv = x_ref[...]
    wh = lax.dot_general(xv, w_ref[...], (((1,), (1,)), ((), ())),
                         preferred_element_type=jnp.float32)
    wh_ref[...] = wh
    s_ref[...] = lax.dot_general(wh, a_ref[...], (((1,), (1,)), ((), ())),
                                 preferred_element_type=jnp.float32)


def _matmul(x, W, A):
    blk = 1000
    grid = N // blk
    return pl.pallas_call(
        _mm_body,
        grid=(grid,),
        in_specs=[
            pl.BlockSpec((blk, D), lambda i: (i, 0)),
            pl.BlockSpec((D, D), lambda i: (0, 0)),
            pl.BlockSpec((8, D), lambda i: (0, 0)),
        ],
        out_specs=[
            pl.BlockSpec((blk, D), lambda i: (i, 0)),
            pl.BlockSpec((blk, 8), lambda i: (i, 0)),
        ],
        out_shape=[
            jax.ShapeDtypeStruct((N, D), jnp.float32),
            jax.ShapeDtypeStruct((N, 8), jnp.float32),
        ],
    )(x, W, A)


def _epi_body(num_ref, den_ref, wh_ref, b_ref, o_ref):
    num = num_ref[0] + num_ref[1]
    den = jnp.sum(den_ref[...], axis=1)
    den = jnp.maximum(den, 1e-9)
    o_ref[...] = jnp.maximum(num / den[:, None] + wh_ref[...] + b_ref[...],
                             0.0)


def _epilogue(num, den, Wh, bias):
    blk = 1000
    grid = N // blk
    return pl.pallas_call(
        _epi_body,
        grid=(grid,),
        in_specs=[
            pl.BlockSpec((2, blk, D), lambda i: (0, i, 0)),
            pl.BlockSpec((blk, NW), lambda i: (i, 0)),
            pl.BlockSpec((blk, D), lambda i: (i, 0)),
            pl.BlockSpec((1, D), lambda i: (0, 0)),
        ],
        out_specs=pl.BlockSpec((blk, D), lambda i: (i, 0)),
        out_shape=jax.ShapeDtypeStruct((N, D), jnp.float32),
    )(num, den, Wh, bias)


def _sc_body(src_hbm, dst_hbm, s1_hbm, s2_hbm, wh_hbm,
             num_out, den_out,
             denom_v, sidx, didx,
             rows0, s1g0, s2g0, rows1, s1g1, s2g1, rows2, s1g2, s2g2,
             eexp_v,
             gsem0, gsem1, gsem2, ssem0, ssem1, ssem2, num_sh):
    cid = lax.axis_index("c")
    sid = lax.axis_index("s")
    wid = cid * NS + sid

    rows = (rows0, rows1, rows2)
    s1g = (s1g0, s1g1, s1g2)
    s2g = (s2g0, s2g1, s2g2)
    gsem = (gsem0, gsem1, gsem2)
    ssem = (ssem0, ssem1, ssem2)

    zv = jnp.zeros((16,), jnp.float32)

    def _zero_denom(r, _):
        denom_v[pl.ds(r * 16, 16)] = zv
        return 0

    lax.fori_loop(0, N_PAD // 16, _zero_denom, 0)

    def _zero_rows(r, _):
        for v in range(D // 16):
            rows0[r, pl.ds(v * 16, 16)] = zv
        return 0

    lax.fori_loop(0, CHUNK, _zero_rows, 0)

    for p in range(ROWS_MAIN // CHUNK):
        pltpu.sync_copy(rows0,
                        num_sh.at[pl.ds(sid * ROWS_MAIN + p * CHUNK, CHUNK)])
    pltpu.sync_copy(
        rows0.at[pl.ds(0, ROWS_MAIN % CHUNK)],
        num_sh.at[pl.ds(sid * ROWS_MAIN + (ROWS_MAIN // CHUNK) * CHUNK,
                        ROWS_MAIN % CHUNK)])

    @pl.when(sid == NS - 1)
    def _zero_rem():
        pltpu.sync_copy(rows0.at[pl.ds(0, REM)],
                        num_sh.at[pl.ds(REM_BASE, REM)])

    plsc.subcore_barrier()

    lane0 = lax.iota(jnp.int32, 16) == 0

    def _issue(c, b):
        pltpu.async_copy(wh_hbm.at[sidx.at[c]], rows[b], gsem[b])
        pltpu.async_copy(s1_hbm.at[sidx.at[c]], s1g[b], gsem[b])
        pltpu.async_copy(s2_hbm.at[didx.at[c]], s2g[b], gsem[b])

    def _wait_gather(b):
        pltpu.make_async_copy(wh_hbm.at[sidx.at[0]], rows[b], gsem[b]).wait()
        pltpu.make_async_copy(s1_hbm.at[sidx.at[0]], s1g[b], gsem[b]).wait()
        pltpu.make_async_copy(s2_hbm.at[didx.at[0]], s2g[b], gsem[b]).wait()

    def _compute(c, b):
        cc = jnp.broadcast_to(c, (16,)).astype(jnp.int32)
        rows_, s1g_, s2g_ = rows[b], s1g[b], s2g[b]
        for g in range(CHUNK // 16):
            sl = pl.ds(g * 16, 16)
            e = s1g_[sl] + s2g_[sl]
            e = jnp.where(e >= 0.0, e, 0.2 * e)
            eexp_v[sl] = jnp.exp(e)

        def _scale(k, _):
            kk = jnp.broadcast_to(k, (16,)).astype(jnp.int32)
            ab = plsc.load_gather(eexp_v, [kk])
            dk = plsc.load_gather(didx, [cc, kk])
            plsc.addupdate_scatter(denom_v, [dk], ab, mask=lane0)
            for v in range(D // 16):
                sl = pl.ds(v * 16, 16)
                rows_[k, sl] = rows_[k, sl] * ab
            return 0

        lax.fori_loop(0, CHUNK, _scale, 0, unroll=8)

    def _step(c, b, prefetch):
        _wait_gather(b)
        _compute(c, b)
        pltpu.async_copy(rows[b], num_sh.at[didx.at[c]], ssem[b], add=True)
        if prefetch:
            bp = (b + 2) % 3

            @pl.when(c == 0)
            def _pf0():
                _issue(2, bp)

            @pl.when((c >= 1) & (c + 2 < CPS))
            def _pf():
                pltpu.make_async_copy(rows[bp], num_sh.at[didx.at[0]],
                                      ssem[bp]).wait()
                _issue(c + 2, bp)

    def _superchunk(s, _):
        g = wid * NSUP + s
        pltpu.sync_copy(src_hbm.at[g], sidx)
        pltpu.sync_copy(dst_hbm.at[g], didx)

        _issue(0, 0)
        _issue(1, 1)

        def _triple(c2, _):
            c = c2 * 3
            for b in range(3):
                _step(c + b, b, True)
            return 0

        lax.fori_loop(0, CPS // 3, _triple, 0)

        _step(CPS - 1, (CPS - 1) % 3, False)
        for b in range(3):
            pltpu.make_async_copy(rows[b], num_sh.at[didx.at[0]],
                                  ssem[b]).wait()
        return 0

    lax.fori_loop(0, NSUP, _superchunk, 0)

    plsc.subcore_barrier()

    r0 = sid * ROWS_MAIN
    pltpu.sync_copy(num_sh.at[pl.ds(r0, ROWS_MAIN)],
                    num_out.at[cid, pl.ds(r0, ROWS_MAIN)])

    @pl.when(sid == NS - 1)
    def _copy_rem():
        pltpu.sync_copy(num_sh.at[pl.ds(REM_BASE, REM)],
                        num_out.at[cid, pl.ds(REM_BASE, REM)])

    pltpu.sync_copy(
        denom_v.at[pl.ds(0, N)],
        den_out.at[pl.ds(pl.multiple_of(wid * N, 8), N)])


def _sc_edge_pass(src, dst, s1, s2, Wh):
    mesh = plsc.VectorSubcoreMesh(core_axis_name="c", subcore_axis_name="s")
    f = pl.kernel(
        _sc_body,
        mesh=mesh,
        compiler_params=pltpu.CompilerParams(needs_layout_passes=False),
        out_type=[
            jax.ShapeDtypeStruct((NC, N, D), jnp.float32),
            jax.ShapeDtypeStruct((NW * N,), jnp.float32),
        ],
        scratch_types=[
            pltpu.VMEM((N_PAD,), jnp.float32),
            pltpu.VMEM((CPS, CHUNK), jnp.int32),
            pltpu.VMEM((CPS, CHUNK), jnp.int32),
            pltpu.VMEM((CHUNK, D), jnp.float32),
            pltpu.VMEM((CHUNK,), jnp.float32),
            pltpu.VMEM((CHUNK,), jnp.float32),
            pltpu.VMEM((CHUNK, D), jnp.float32),
            pltpu.VMEM((CHUNK,), jnp.float32),
            pltpu.VMEM((CHUNK,), jnp.float32),
            pltpu.VMEM((CHUNK, D), jnp.float32),
            pltpu.VMEM((CHUNK,), jnp.float32),
            pltpu.VMEM((CHUNK,), jnp.float32),
            pltpu.VMEM((CHUNK,), jnp.float32),
            pltpu.SemaphoreType.DMA,
            pltpu.SemaphoreType.DMA,
            pltpu.SemaphoreType.DMA,
            pltpu.SemaphoreType.DMA,
            pltpu.SemaphoreType.DMA,
            pltpu.SemaphoreType.DMA,
            pltpu.VMEM_SHARED((N_PAD, D), jnp.float32),
        ],
    )
    return f(src, dst, s1, s2, Wh)


def kernel(x, edge_index, W, a_w, bias):
    src_r = edge_index[0].reshape(NW * NSUP, CPS, CHUNK)
    dst_r = edge_index[1].reshape(NW * NSUP, CPS, CHUNK)
    A = jnp.zeros((8, D), jnp.float32)
    A = A.at[0].set(a_w[0, :D]).at[1].set(a_w[0, D:])
    Wh, s = _matmul(x, W, A)
    num, den = _sc_edge_pass(src_r, dst_r, s[:, 0], s[:, 1], Wh)
    return _epilogue(num, den.reshape(NW, N).T, Wh, bias)


# speedup vs baseline: 1.8819x; 1.8819x over previous
import jax
import jax.numpy as jnp
from jax import lax
from jax.experimental import pallas as pl
from jax.experimental.pallas import tpu as pltpu
from jax.experimental.pallas import tpu_sc as plsc

N = 10000
E = 320000
D = 128

NC = 2
NS = 16
NW = NC * NS

CHUNK = 80
CPS = 25
NSUP = 5
EDGES_PER_TILE = CHUNK * CPS * NSUP
N_PAD = N
ROWS_MAIN = 624
REM_BASE = NS * ROWS_MAIN
REM = N - REM_BASE


def _mm_body(x_ref, w_ref, a_ref, wh_ref, s_ref):
    xv = x_ref[...]
    wh = lax.dot_general(xv, w_ref[...], (((1,), (1,)), ((), ())),
                         preferred_element_type=jnp.float32)
    wh_ref[...] = wh
    s_ref[...] = lax.dot_general(wh, a_ref[...], (((1,), (1,)), ((), ())),
                                 preferred_element_type=jnp.float32)


def _matmul(x, W, A):
    blk = 1000
    grid = N // blk
    return pl.pallas_call(
        _mm_body,
        grid=(grid,),
        in_specs=[
            pl.BlockSpec((blk, D), lambda i: (i, 0)),
            pl.BlockSpec((D, D), lambda i: (0, 0)),
            pl.BlockSpec((8, D), lambda i: (0, 0)),
        ],
        out_specs=[
            pl.BlockSpec((blk, D), lambda i: (i, 0)),
            pl.BlockSpec((blk, 8), lambda i: (i, 0)),
        ],
        out_shape=[
            jax.ShapeDtypeStruct((N, D), jnp.float32),
            jax.ShapeDtypeStruct((N, 8), jnp.float32),
        ],
    )(x, W, A)


def _epi_body(num_ref, den_ref, wh_ref, b_ref, o_ref):
    num = num_ref[0] + num_ref[1]
    den = jnp.sum(den_ref[...], axis=1)
    den = jnp.maximum(den, 1e-9)
    o_ref[...] = jnp.maximum(num / den[:, None] + wh_ref[...] + b_ref[...],
                             0.0)


def _epilogue(num, den, Wh, bias):
    blk = 1000
    grid = N // blk
    return pl.pallas_call(
        _epi_body,
        grid=(grid,),
        in_specs=[
            pl.BlockSpec((2, blk, D), lambda i: (0, i, 0)),
            pl.BlockSpec((blk, NW), lambda i: (i, 0)),
            pl.BlockSpec((blk, D), lambda i: (i, 0)),
            pl.BlockSpec((1, D), lambda i: (0, 0)),
        ],
        out_specs=pl.BlockSpec((blk, D), lambda i: (i, 0)),
        out_shape=jax.ShapeDtypeStruct((N, D), jnp.float32),
    )(num, den, Wh, bias)


def _sc_body(src_hbm, dst_hbm, s1_hbm, s2_hbm, wh_hbm,
             num_out, den_out,
             denom_v, sidx, didx,
             rows0, s1g0, s2g0, rows1, s1g1, s2g1, rows2, s1g2, s2g2,
             eexp_v,
             gsem0, gsem1, gsem2, ssem0, ssem1, ssem2, num_sh):
    cid = lax.axis_index("c")
    sid = lax.axis_index("s")
    wid = cid * NS + sid

    rows = (rows0, rows1, rows2)
    s1g = (s1g0, s1g1, s1g2)
    s2g = (s2g0, s2g1, s2g2)
    gsem = (gsem0, gsem1, gsem2)
    ssem = (ssem0, ssem1, ssem2)

    zv = jnp.zeros((16,), jnp.float32)

    def _zero_denom(r, _):
        denom_v[pl.ds(r * 16, 16)] = zv
        return 0

    lax.fori_loop(0, N_PAD // 16, _zero_denom, 0)

    def _zero_rows(r, _):
        for v in range(D // 16):
            rows0[r, pl.ds(v * 16, 16)] = zv
        return 0

    lax.fori_loop(0, CHUNK, _zero_rows, 0)

    for p in range(ROWS_MAIN // CHUNK):
        pltpu.sync_copy(rows0,
                        num_sh.at[pl.ds(sid * ROWS_MAIN + p * CHUNK, CHUNK)])
    pltpu.sync_copy(
        rows0.at[pl.ds(0, ROWS_MAIN % CHUNK)],
        num_sh.at[pl.ds(sid * ROWS_MAIN + (ROWS_MAIN // CHUNK) * CHUNK,
                        ROWS_MAIN % CHUNK)])

    @pl.when(sid == NS - 1)
    def _zero_rem():
        pltpu.sync_copy(rows0.at[pl.ds(0, REM)],
                        num_sh.at[pl.ds(REM_BASE, REM)])

    plsc.subcore_barrier()

    lane0 = lax.iota(jnp.int32, 16) == 0

    def _issue(c, b):
        pltpu.async_copy(wh_hbm.at[sidx.at[c]], rows[b], gsem[b])
        pltpu.async_copy(s1_hbm.at[sidx.at[c]], s1g[b], gsem[b])
        pltpu.async_copy(s2_hbm.at[didx.at[c]], s2g[b], gsem[b])

    def _wait_gather(b):
        pltpu.make_async_copy(wh_hbm.at[sidx.at[0]], rows[b], gsem[b]).wait()
        pltpu.make_async_copy(s1_hbm.at[sidx.at[0]], s1g[b], gsem[b]).wait()
        pltpu.make_async_copy(s2_hbm.at[didx.at[0]], s2g[b], gsem[b]).wait()

    def _compute(c, b):
        cc = jnp.broadcast_to(c, (16,)).astype(jnp.int32)
        rows_, s1g_, s2g_ = rows[b], s1g[b], s2g[b]
        for g in range(CHUNK // 16):
            sl = pl.ds(g * 16, 16)
            e = s1g_[sl] + s2g_[sl]
            e = jnp.where(e >= 0.0, e, 0.2 * e)
            eexp_v[sl] = jnp.exp(e)

        def _scale(k, _):
            kk = jnp.broadcast_to(k, (16,)).astype(jnp.int32)
            ab = plsc.load_gather(eexp_v, [kk])
            dk = plsc.load_gather(didx, [cc, kk])
            plsc.addupdate_scatter(denom_v, [dk], ab, mask=lane0)
            for v in range(D // 16):
                sl = pl.ds(v * 16, 16)
                rows_[k, sl] = rows_[k, sl] * ab
            return 0

        lax.fori_loop(0, CHUNK, _scale, 0, unroll=4)

    def _step(c, b, prefetch):
        _wait_gather(b)
        _compute(c, b)
        pltpu.async_copy(rows[b], num_sh.at[didx.at[c]], ssem[b], add=True)
        if prefetch:
            bp = (b + 2) % 3

            @pl.when(c == 0)
            def _pf0():
                _issue(2, bp)

            @pl.when((c >= 1) & (c + 2 < CPS))
            def _pf():
                pltpu.make_async_copy(rows[bp], num_sh.at[didx.at[0]],
                                      ssem[bp]).wait()
                _issue(c + 2, bp)

    def _superchunk(s, _):
        g = wid * NSUP + s
        pltpu.sync_copy(src_hbm.at[g], sidx)
        pltpu.sync_copy(dst_hbm.at[g], didx)

        _issue(0, 0)
        _issue(1, 1)

        def _triple(c2, _):
            c = c2 * 3
            for b in range(3):
                _step(c + b, b, True)
            return 0

        lax.fori_loop(0, CPS // 3, _triple, 0)

        _step(CPS - 1, (CPS - 1) % 3, False)
        for b in range(3):
            pltpu.make_async_copy(rows[b], num_sh.at[didx.at[0]],
                                  ssem[b]).wait()
        return 0

    lax.fori_loop(0, NSUP, _superchunk, 0)

    plsc.subcore_barrier()

    r0 = sid * ROWS_MAIN
    pltpu.sync_copy(num_sh.at[pl.ds(r0, ROWS_MAIN)],
                    num_out.at[cid, pl.ds(r0, ROWS_MAIN)])

    @pl.when(sid == NS - 1)
    def _copy_rem():
        pltpu.sync_copy(num_sh.at[pl.ds(REM_BASE, REM)],
                        num_out.at[cid, pl.ds(REM_BASE, REM)])

    pltpu.sync_copy(
        denom_v.at[pl.ds(0, N)],
        den_out.at[pl.ds(pl.multiple_of(wid * N, 8), N)])


def _sc_edge_pass(src, dst, s1, s2, Wh):
    mesh = plsc.VectorSubcoreMesh(core_axis_name="c", subcore_axis_name="s")
    f = pl.kernel(
        _sc_body,
        mesh=mesh,
        compiler_params=pltpu.CompilerParams(needs_layout_passes=False),
        out_type=[
            jax.ShapeDtypeStruct((NC, N, D), jnp.float32),
            jax.ShapeDtypeStruct((NW * N,), jnp.float32),
        ],
        scratch_types=[
            pltpu.VMEM((N_PAD,), jnp.float32),
            pltpu.VMEM((CPS, CHUNK), jnp.int32),
            pltpu.VMEM((CPS, CHUNK), jnp.int32),
            pltpu.VMEM((CHUNK, D), jnp.float32),
            pltpu.VMEM((CHUNK,), jnp.float32),
            pltpu.VMEM((CHUNK,), jnp.float32),
            pltpu.VMEM((CHUNK, D), jnp.float32),
            pltpu.VMEM((CHUNK,), jnp.float32),
            pltpu.VMEM((CHUNK,), jnp.float32),
            pltpu.VMEM((CHUNK, D), jnp.float32),
            pltpu.VMEM((CHUNK,), jnp.float32),
            pltpu.VMEM((CHUNK,), jnp.float32),
            pltpu.VMEM((CHUNK,), jnp.float32),
            pltpu.SemaphoreType.DMA,
            pltpu.SemaphoreType.DMA,
            pltpu.SemaphoreType.DMA,
            pltpu.SemaphoreType.DMA,
            pltpu.SemaphoreType.DMA,
            pltpu.SemaphoreType.DMA,
            pltpu.VMEM_SHARED((N_PAD, D), jnp.float32),
        ],
    )
    return f(src, dst, s1, s2, Wh)


def kernel(x, edge_index, W, a_w, bias):
    src_r = edge_index[0].reshape(NW * NSUP, CPS, CHUNK)
    dst_r = edge_index[1].reshape(NW * NSUP, CPS, CHUNK)
    A = jnp.zeros((8, D), jnp.float32)
    A = A.at[0].set(a_w[0, :D]).at[1].set(a_w[0, D:])
    Wh, s = _matmul(x, W, A)
    num, den = _sc_edge_pass(src_r, dst_r, s[:, 0], s[:, 1], Wh)
    return _epilogue(num, den.reshape(NW, N).T, Wh, bias)
